# baseline (speedup 1.0000x reference)
import jax
import jax.numpy as jnp
from jax import lax
from jax.experimental import pallas as pl
from jax.experimental.pallas import tpu as pltpu

B = 1024
C = 3
S = 12
P = 32
D = P * P

NSLAB = C * S
NBUF = 8
TB = 128
GITER = 12
GPJ = -(-B // GITER)
TPOSE_AT = GITER + 1


def _body(x_hbm, idx_smem, table_hbm, out_hbm, xbuf, rows, rows_t,
          sem_g, sem_r, sem_in, sem_out):
    def copy_in(i):
        return pltpu.make_async_copy(
            x_hbm.at[i // S, i % S], xbuf.at[i % NBUF], sem_in)

    def copy_out(i):
        return pltpu.make_async_copy(
            xbuf.at[i % NBUF], out_hbm.at[i // S, i % S], sem_out)

    for b in range(NBUF):
        copy_in(b).start()

    def issue(i, carry):
        r = idx_smem[i]
        pltpu.make_async_copy(
            table_hbm.at[pl.ds(r, 1)], rows.at[pl.ds(i, 1)], sem_g
        ).start()
        return carry

    for j in range(NSLAB):
        if j < GITER:
            lax.fori_loop(j * GPJ, min((j + 1) * GPJ, B), issue, 0)
        if j == TPOSE_AT:
            pltpu.make_async_copy(table_hbm.at[pl.ds(0, B)], rows, sem_g).wait()
            for ib in range(B // TB):
                for jb in range(D // TB):
                    t = rows[pl.ds(ib * TB, TB), pl.ds(jb * TB, TB)]
                    rows_t[pl.ds(jb * TB, TB), pl.ds(ib * TB, TB)] = t.T
            for s in range(S):
                pltpu.make_async_copy(rows_t, out_hbm.at[C, s], sem_r).start()
        if j >= 1:
            copy_out(j - 1).wait()
            nxt = j - 1 + NBUF
            if nxt < NSLAB:
                copy_in(nxt).start()
        copy_in(j).wait()
        copy_out(j).start()
    copy_out(NSLAB - 1).wait()

    for s in range(S):
        pltpu.make_async_copy(rows_t, out_hbm.at[C, s], sem_r).wait()


def kernel(x, id, table):
    x_t = jnp.transpose(x.reshape(B, C, S, D), (1, 2, 3, 0))
    out_t = pl.pallas_call(
        _body,
        out_shape=jax.ShapeDtypeStruct((C + 1, S, D, B), jnp.float32),
        in_specs=[
            pl.BlockSpec(memory_space=pl.MemorySpace.ANY),
            pl.BlockSpec(memory_space=pltpu.SMEM),
            pl.BlockSpec(memory_space=pl.MemorySpace.ANY),
        ],
        out_specs=pl.BlockSpec(memory_space=pl.MemorySpace.ANY),
        scratch_shapes=[
            pltpu.VMEM((NBUF, D, B), jnp.float32),
            pltpu.VMEM((B, D), jnp.float32),
            pltpu.VMEM((D, B), jnp.float32),
            pltpu.SemaphoreType.DMA,
            pltpu.SemaphoreType.DMA,
            pltpu.SemaphoreType.DMA,
            pltpu.SemaphoreType.DMA,
        ],
    )(x_t, id, table)
    return jnp.transpose(out_t, (3, 0, 1, 2)).reshape(B, C + 1, S, P, P)


# speedup vs baseline: 1.0080x; 1.0080x over previous
import jax
import jax.numpy as jnp
from jax import lax
from jax.experimental import pallas as pl
from jax.experimental.pallas import tpu as pltpu

B = 1024
C = 3
S = 12
P = 32
D = P * P

NSLAB = C * S
NBUF = 6
TB = 128
GITER = 12
GPJ = -(-B // GITER)
TPOSE_AT = GITER + 1


def _body(x_hbm, idx_smem, table_hbm, out_hbm, xbuf, rows, rows_t,
          sem_g, sem_r, sem_in, sem_out):
    def copy_in(i):
        return pltpu.make_async_copy(
            x_hbm.at[i // S, i % S], xbuf.at[i % NBUF], sem_in.at[i % 2])

    def copy_out(i):
        return pltpu.make_async_copy(
            xbuf.at[i % NBUF], out_hbm.at[i // S, i % S], sem_out.at[i % 2])

    for b in range(NBUF):
        copy_in(b).start()

    def issue(i, carry):
        r = idx_smem[i]
        pltpu.make_async_copy(
            table_hbm.at[pl.ds(r, 1)], rows.at[pl.ds(i, 1)], sem_g
        ).start()
        return carry

    for j in range(NSLAB):
        if j < GITER:
            lax.fori_loop(j * GPJ, min((j + 1) * GPJ, B), issue, 0)
        if j == TPOSE_AT:
            pltpu.make_async_copy(table_hbm.at[pl.ds(0, B)], rows, sem_g).wait()
            for ib in range(B // TB):
                for jb in range(D // TB):
                    t = rows[pl.ds(ib * TB, TB), pl.ds(jb * TB, TB)]
                    rows_t[pl.ds(jb * TB, TB), pl.ds(ib * TB, TB)] = t.T
            for s in range(S):
                pltpu.make_async_copy(rows_t, out_hbm.at[C, s], sem_r).start()
        if j >= 1:
            copy_out(j - 1).wait()
            nxt = j - 1 + NBUF
            if nxt < NSLAB:
                copy_in(nxt).start()
        copy_in(j).wait()
        copy_out(j).start()
    copy_out(NSLAB - 1).wait()

    for s in range(S):
        pltpu.make_async_copy(rows_t, out_hbm.at[C, s], sem_r).wait()


def kernel(x, id, table):
    x_t = jnp.transpose(x.reshape(B, C, S, D), (1, 2, 3, 0))
    out_t = pl.pallas_call(
        _body,
        out_shape=jax.ShapeDtypeStruct((C + 1, S, D, B), jnp.float32),
        in_specs=[
            pl.BlockSpec(memory_space=pl.MemorySpace.ANY),
            pl.BlockSpec(memory_space=pltpu.SMEM),
            pl.BlockSpec(memory_space=pl.MemorySpace.ANY),
        ],
        out_specs=pl.BlockSpec(memory_space=pl.MemorySpace.ANY),
        scratch_shapes=[
            pltpu.VMEM((NBUF, D, B), jnp.float32),
            pltpu.VMEM((B, D), jnp.float32),
            pltpu.VMEM((D, B), jnp.float32),
            pltpu.SemaphoreType.DMA,
            pltpu.SemaphoreType.DMA,
            pltpu.SemaphoreType.DMA((2,)),
            pltpu.SemaphoreType.DMA((2,)),
        ],
    )(x_t, id, table)
    return jnp.transpose(out_t, (3, 0, 1, 2)).reshape(B, C + 1, S, P, P)
